# baseline (speedup 1.0000x reference)
import functools

import jax
import jax.numpy as jnp
from jax import lax
from jax.experimental import pallas as pl
from jax.experimental.pallas import tpu as pltpu
from jax.experimental.pallas import tpu_sc as plsc

NN = 10000
EE = 320000
HH = 128
HF = 64
CC = 10
GG = 64

NC = 2
NS = 16
EPT = EE // NS
CK = 80
NCH = EPT // CK
NBUF = 2
RPT = NN // NS

NH = NN // 2
DROWS = NH + 8
DRPT = DROWS // NS

MBLK = 1000
NBLK = NN // MBLK

_mesh = plsc.VectorSubcoreMesh(core_axis_name="c", subcore_axis_name="s")


@functools.partial(
    pl.kernel,
    mesh=_mesh,
    compiler_params=pltpu.CompilerParams(use_tc_tiling_on_sc=False),
    out_type=jax.ShapeDtypeStruct((NC, NS, DRPT, 16), jnp.float32),
    scratch_types=[
        pltpu.VMEM((CK,), jnp.int32),
        pltpu.VMEM((CK, 16), jnp.float32),
        pltpu.VMEM((DRPT, 16), jnp.float32),
        pltpu.VMEM_SHARED((DROWS, 16), jnp.float32),
    ],
)
def _sc_degree(rdst_hbm, ones_hbm, zeros_hbm, out_hbm, didx, ones_v, stage, acc):
    c = lax.axis_index("c")
    s = lax.axis_index("s")
    pltpu.sync_copy(zeros_hbm, stage)
    pltpu.sync_copy(stage, acc.at[pl.ds(s * DRPT, DRPT)])
    pltpu.sync_copy(ones_hbm, ones_v)
    plsc.subcore_barrier()

    def body(j, carry):
        pltpu.sync_copy(rdst_hbm.at[c, s, j], didx)
        pltpu.sync_copy(ones_v, acc.at[didx], add=True)
        return carry

    lax.fori_loop(0, NCH, body, 0)
    plsc.subcore_barrier()
    pltpu.sync_copy(acc.at[pl.ds(s * DRPT, DRPT)], stage)
    pltpu.sync_copy(stage, out_hbm.at[c, s])


@functools.partial(
    pl.kernel,
    mesh=_mesh,
    compiler_params=pltpu.CompilerParams(use_tc_tiling_on_sc=False),
    out_type=jax.ShapeDtypeStruct((NN, NC, HF), jnp.float32),
    scratch_types=[
        pltpu.VMEM((EPT,), jnp.int32),
        pltpu.VMEM((NCH, CK), jnp.int32),
    ] + [pltpu.VMEM((CK, HF), jnp.float32)] * NBUF
      + [
        pltpu.VMEM((RPT, HF), jnp.float32),
        pltpu.VMEM_SHARED((NN, HF), jnp.float32),
    ] + [pltpu.SemaphoreType.DMA] * (2 * NBUF),
)
def _sc_aggregate(q_hbm, gsrc_hbm, dst_hbm, zeros_hbm, out_hbm,
                  gidx, didx, *bufs_and_sems):
    rows = bufs_and_sems[:NBUF]
    stage = bufs_and_sems[NBUF]
    acc = bufs_and_sems[NBUF + 1]
    gsem = bufs_and_sems[NBUF + 2:2 * NBUF + 2]
    ssem = bufs_and_sems[2 * NBUF + 2:]
    c = lax.axis_index("c")
    s = lax.axis_index("s")
    pltpu.sync_copy(zeros_hbm, stage)
    pltpu.sync_copy(stage, acc.at[pl.ds(s * RPT, RPT)])
    pltpu.sync_copy(gsrc_hbm.at[c, s], gidx)
    pltpu.sync_copy(dst_hbm.at[s], didx)
    for b in range(NBUF):
        pltpu.async_copy(q_hbm.at[gidx.at[pl.ds(b * CK, CK)]], rows[b], gsem[b])
    plsc.subcore_barrier()

    def body(k, carry):
        base = NBUF * k
        for b in range(NBUF):
            pltpu.make_async_copy(q_hbm.at[pl.ds(0, CK)], rows[b], gsem[b]).wait()
            pltpu.async_copy(rows[b], acc.at[didx.at[base + b]], ssem[b],
                             add=True)
        for b in range(NBUF):
            pltpu.make_async_copy(zeros_hbm.at[pl.ds(0, CK)], rows[b],
                                  ssem[b]).wait()
            jn = jnp.minimum((base + NBUF + b) * CK, (NCH - 1) * CK)
            pltpu.async_copy(q_hbm.at[gidx.at[pl.ds(jn, CK)]], rows[b], gsem[b])
        return carry

    lax.fori_loop(0, NCH // NBUF, body, 0)
    for b in range(NBUF):
        pltpu.make_async_copy(q_hbm.at[pl.ds(0, CK)], rows[b], gsem[b]).wait()
    plsc.subcore_barrier()
    pltpu.sync_copy(acc.at[pl.ds(s * RPT, RPT)], stage)
    pltpu.sync_copy(stage, out_hbm.at[pl.ds(s * RPT, RPT), c])


def _tc_first_body(cnt_ref, x_ref, w_ref, q_ref, dinv_ref):
    deg = cnt_ref[...][:, :1] + 1.0
    dinv = lax.rsqrt(deg)
    p = jnp.dot(x_ref[...], w_ref[...], preferred_element_type=jnp.float32)
    q_ref[...] = p * dinv
    dinv_ref[...] = dinv


_tc_first = pl.pallas_call(
    _tc_first_body,
    grid=(NBLK,),
    in_specs=[
        pl.BlockSpec((MBLK, 16), lambda i: (i, 0)),
        pl.BlockSpec((MBLK, HH), lambda i: (i, 0)),
        pl.BlockSpec((HH, HH), lambda i: (0, 0)),
    ],
    out_specs=[
        pl.BlockSpec((MBLK, HH), lambda i: (i, 0)),
        pl.BlockSpec((MBLK, 1), lambda i: (i, 0)),
    ],
    out_shape=[
        jax.ShapeDtypeStruct((NN, HH), jnp.float32),
        jax.ShapeDtypeStruct((NN, 1), jnp.float32),
    ],
)


def _tc_mid_body(acc_ref, q_ref, dinv_ref, b_ref, w_ref, qn_ref):
    z = (acc_ref[...] + q_ref[...]) * dinv_ref[...] + b_ref[...]
    h = jnp.maximum(z, 0.0)
    qn = jnp.dot(h, w_ref[...], preferred_element_type=jnp.float32)
    qn_ref[...] = qn * dinv_ref[...]


_tc_mid = pl.pallas_call(
    _tc_mid_body,
    grid=(NBLK,),
    in_specs=[
        pl.BlockSpec((MBLK, HH), lambda i: (i, 0)),
        pl.BlockSpec((MBLK, HH), lambda i: (i, 0)),
        pl.BlockSpec((MBLK, 1), lambda i: (i, 0)),
        pl.BlockSpec((1, HH), lambda i: (0, 0)),
        pl.BlockSpec((HH, HH), lambda i: (0, 0)),
    ],
    out_specs=pl.BlockSpec((MBLK, HH), lambda i: (i, 0)),
    out_shape=jax.ShapeDtypeStruct((NN, HH), jnp.float32),
)


def _tc_final_body(acc_ref, q_ref, dinv_ref, b_ref, batch_ref, wl_ref, bl_ref,
                   out_ref, pool_scr, cnt_scr):
    i = pl.program_id(0)

    @pl.when(i == 0)
    def _init():
        pool_scr[...] = jnp.zeros_like(pool_scr)
        cnt_scr[...] = jnp.zeros_like(cnt_scr)

    z = (acc_ref[...] + q_ref[...]) * dinv_ref[...] + b_ref[...]
    h = jnp.maximum(z, 0.0)
    bb = batch_ref[0, 0, :]
    gids = lax.broadcasted_iota(jnp.int32, (GG, MBLK), 0)
    oh = (bb[None, :] == gids).astype(jnp.float32)
    pool_scr[...] += jnp.dot(oh, h, preferred_element_type=jnp.float32)
    cnt_scr[...] += jnp.sum(oh, axis=1, keepdims=True)

    @pl.when(i == NBLK - 1)
    def _fin():
        pooled = pool_scr[...] / jnp.maximum(cnt_scr[...], 1.0)
        out_ref[...] = (
            jnp.dot(pooled, wl_ref[...], preferred_element_type=jnp.float32)
            + bl_ref[...]
        )


_tc_final = pl.pallas_call(
    _tc_final_body,
    grid=(NBLK,),
    in_specs=[
        pl.BlockSpec((MBLK, HH), lambda i: (i, 0)),
        pl.BlockSpec((MBLK, HH), lambda i: (i, 0)),
        pl.BlockSpec((MBLK, 1), lambda i: (i, 0)),
        pl.BlockSpec((1, HH), lambda i: (0, 0)),
        pl.BlockSpec((1, 1, MBLK), lambda i: (i, 0, 0)),
        pl.BlockSpec((HH, CC), lambda i: (0, 0)),
        pl.BlockSpec((1, CC), lambda i: (0, 0)),
    ],
    out_specs=pl.BlockSpec((GG, CC), lambda i: (0, 0)),
    out_shape=jax.ShapeDtypeStruct((GG, CC), jnp.float32),
    scratch_shapes=[
        pltpu.VMEM((GG, HH), jnp.float32),
        pltpu.VMEM((GG, 1), jnp.float32),
    ],
)


def kernel(x, edge_index, batch, W1, b1, W2, b2, W3, b3, Wl, bl):
    src = edge_index[0]
    dst = edge_index[1]
    gsrc = jnp.stack([src * 2, src * 2 + 1]).reshape(NC, NS, EPT)
    dst_c = dst.reshape(NS, NCH, CK)
    in0 = dst < NH
    r0 = jnp.where(in0, dst, NH)
    r1 = jnp.where(in0, NH, dst - NH)
    rdst = jnp.stack([r0, r1]).reshape(NC, NS, NCH, CK)

    zeros_h = jnp.zeros((RPT, HF), jnp.float32)
    zeros_d = jnp.zeros((DRPT, 16), jnp.float32)
    ones_d = jnp.ones((CK, 16), jnp.float32)

    deg = _sc_degree(rdst, ones_d, zeros_d)
    deg = deg.reshape(NC, DROWS, 16)
    cnt = jnp.concatenate([deg[0, :NH], deg[1, :NH]], axis=0)
    q1, dinv = _tc_first(cnt, x, W1)
    a1 = _sc_aggregate(q1.reshape(2 * NN, HF), gsrc, dst_c, zeros_h)
    q2 = _tc_mid(a1.reshape(NN, HH), q1, dinv, b1.reshape(1, HH), W2)
    a2 = _sc_aggregate(q2.reshape(2 * NN, HF), gsrc, dst_c, zeros_h)
    q3 = _tc_mid(a2.reshape(NN, HH), q2, dinv, b2.reshape(1, HH), W3)
    a3 = _sc_aggregate(q3.reshape(2 * NN, HF), gsrc, dst_c, zeros_h)
    out = _tc_final(a3.reshape(NN, HH), q3, dinv, b3.reshape(1, HH),
                    batch.reshape(NBLK, 1, MBLK), Wl, bl.reshape(1, CC))
    return out


# speedup vs baseline: 1.1132x; 1.1132x over previous
import functools

import jax
import jax.numpy as jnp
from jax import lax
from jax.experimental import pallas as pl
from jax.experimental.pallas import tpu as pltpu
from jax.experimental.pallas import tpu_sc as plsc

NN = 10000
EE = 320000
HH = 128
HF = 64
CC = 10
GG = 64

NC = 2
NS = 16
EPT = EE // NS
CK = 80
NCH = EPT // CK
EPTP = NCH * CK
DDEP = 5
DCK = 80
DNCH = EPT // DCK

ARW = NN
RPT = ARW // NS

NH = NN // 2
DROWS = NH + 8
DRPT = DROWS // NS

MBLK = 1000
NBLK = NN // MBLK

_mesh = plsc.VectorSubcoreMesh(core_axis_name="c", subcore_axis_name="s")


@functools.partial(
    pl.kernel,
    mesh=_mesh,
    compiler_params=pltpu.CompilerParams(use_tc_tiling_on_sc=False),
    out_type=jax.ShapeDtypeStruct((NC, NS, DRPT, 16), jnp.float32),
    scratch_types=[
        pltpu.VMEM((DNCH, DCK), jnp.int32),
        pltpu.VMEM((DCK, 16), jnp.float32),
        pltpu.VMEM((DRPT, 16), jnp.float32),
        pltpu.VMEM_SHARED((DROWS, 16), jnp.float32),
    ] + [pltpu.SemaphoreType.DMA] * DDEP,
)
def _sc_degree(rdst_hbm, ones_hbm, zeros_hbm, out_hbm, didx, ones_v, stage, acc,
               *sems):
    c = lax.axis_index("c")
    s = lax.axis_index("s")
    pltpu.sync_copy(zeros_hbm, stage)
    pltpu.sync_copy(stage, acc.at[pl.ds(s * DRPT, DRPT)])
    pltpu.sync_copy(ones_hbm, ones_v)
    pltpu.sync_copy(rdst_hbm.at[c, s], didx)
    plsc.subcore_barrier()

    def body(k, carry):
        for b in range(DDEP):
            @pl.when(k > 0)
            def _drain():
                pltpu.make_async_copy(ones_hbm, ones_v, sems[b]).wait()

            pltpu.async_copy(ones_v, acc.at[didx.at[DDEP * k + b]], sems[b],
                             add=True)
        return carry

    lax.fori_loop(0, DNCH // DDEP, body, 0)
    for b in range(DDEP):
        pltpu.make_async_copy(ones_hbm, ones_v, sems[b]).wait()
    plsc.subcore_barrier()
    pltpu.sync_copy(acc.at[pl.ds(s * DRPT, DRPT)], stage)
    pltpu.sync_copy(stage, out_hbm.at[c, s])


@functools.partial(
    pl.kernel,
    mesh=_mesh,
    compiler_params=pltpu.CompilerParams(use_tc_tiling_on_sc=False),
    out_type=jax.ShapeDtypeStruct((ARW, NC, HF), jnp.float32),
    scratch_types=[
        pltpu.VMEM((EPTP,), jnp.int32),
        pltpu.VMEM((NCH, CK), jnp.int32),
        pltpu.VMEM((CK, HF), jnp.float32),
        pltpu.VMEM((CK, HF), jnp.float32),
        pltpu.VMEM((RPT, HF), jnp.float32),
        pltpu.VMEM_SHARED((ARW, HF), jnp.float32),
        pltpu.SemaphoreType.DMA,
        pltpu.SemaphoreType.DMA,
    ],
)
def _sc_aggregate(q_hbm, gsrc_hbm, dst_hbm, zeros_hbm, out_hbm,
                  gidx, didx, rows0, rows1, stage, acc, g0, g1):
    c = lax.axis_index("c")
    s = lax.axis_index("s")
    pltpu.sync_copy(zeros_hbm, stage)
    pltpu.sync_copy(stage, acc.at[pl.ds(s * RPT, RPT)])
    pltpu.sync_copy(gsrc_hbm.at[c, s], gidx)
    pltpu.sync_copy(dst_hbm.at[s], didx)
    pltpu.async_copy(q_hbm.at[gidx.at[pl.ds(0, CK)]], rows0, g0)
    plsc.subcore_barrier()

    def body(i2, carry):
        i0 = 2 * i2
        i1 = i0 + 1
        h1 = pltpu.async_copy(q_hbm.at[gidx.at[pl.ds(i1 * CK, CK)]], rows1, g1)
        pltpu.make_async_copy(q_hbm.at[pl.ds(0, CK)], rows0, g0).wait()
        pltpu.sync_copy(rows0, acc.at[didx.at[i0]], add=True)
        inext = jnp.minimum((i0 + 2) * CK, (NCH - 1) * CK)
        pltpu.async_copy(q_hbm.at[gidx.at[pl.ds(inext, CK)]], rows0, g0)
        h1.wait()
        pltpu.sync_copy(rows1, acc.at[didx.at[i1]], add=True)
        return carry

    lax.fori_loop(0, NCH // 2, body, 0)
    pltpu.make_async_copy(q_hbm.at[pl.ds(0, CK)], rows0, g0).wait()
    plsc.subcore_barrier()
    pltpu.sync_copy(acc.at[pl.ds(s * RPT, RPT)], stage)
    pltpu.sync_copy(stage, out_hbm.at[pl.ds(s * RPT, RPT), c])


def _tc_first_body(cnt_ref, x_ref, w_ref, q_ref, dinv_ref):
    deg = cnt_ref[...][:, :1] + 1.0
    dinv = lax.rsqrt(deg)
    p = jnp.dot(x_ref[...], w_ref[...], preferred_element_type=jnp.float32)
    q_ref[...] = p * dinv
    dinv_ref[...] = dinv


_tc_first = pl.pallas_call(
    _tc_first_body,
    grid=(NBLK,),
    in_specs=[
        pl.BlockSpec((MBLK, 16), lambda i: (i, 0)),
        pl.BlockSpec((MBLK, HH), lambda i: (i, 0)),
        pl.BlockSpec((HH, HH), lambda i: (0, 0)),
    ],
    out_specs=[
        pl.BlockSpec((MBLK, HH), lambda i: (i, 0)),
        pl.BlockSpec((MBLK, 1), lambda i: (i, 0)),
    ],
    out_shape=[
        jax.ShapeDtypeStruct((NN, HH), jnp.float32),
        jax.ShapeDtypeStruct((NN, 1), jnp.float32),
    ],
)


def _tc_mid_body(acc_ref, q_ref, dinv_ref, b_ref, w_ref, qn_ref):
    z = (acc_ref[...] + q_ref[...]) * dinv_ref[...] + b_ref[...]
    h = jnp.maximum(z, 0.0)
    qn = jnp.dot(h, w_ref[...], preferred_element_type=jnp.float32)
    qn_ref[...] = qn * dinv_ref[...]


_tc_mid = pl.pallas_call(
    _tc_mid_body,
    grid=(NBLK,),
    in_specs=[
        pl.BlockSpec((MBLK, HH), lambda i: (i, 0)),
        pl.BlockSpec((MBLK, HH), lambda i: (i, 0)),
        pl.BlockSpec((MBLK, 1), lambda i: (i, 0)),
        pl.BlockSpec((1, HH), lambda i: (0, 0)),
        pl.BlockSpec((HH, HH), lambda i: (0, 0)),
    ],
    out_specs=pl.BlockSpec((MBLK, HH), lambda i: (i, 0)),
    out_shape=jax.ShapeDtypeStruct((NN, HH), jnp.float32),
)


def _tc_final_body(acc_ref, q_ref, dinv_ref, b_ref, batch_ref, wl_ref, bl_ref,
                   out_ref, pool_scr, cnt_scr):
    i = pl.program_id(0)

    @pl.when(i == 0)
    def _init():
        pool_scr[...] = jnp.zeros_like(pool_scr)
        cnt_scr[...] = jnp.zeros_like(cnt_scr)

    z = (acc_ref[...] + q_ref[...]) * dinv_ref[...] + b_ref[...]
    h = jnp.maximum(z, 0.0)
    bb = batch_ref[0, 0, :]
    gids = lax.broadcasted_iota(jnp.int32, (GG, MBLK), 0)
    oh = (bb[None, :] == gids).astype(jnp.float32)
    pool_scr[...] += jnp.dot(oh, h, preferred_element_type=jnp.float32)
    cnt_scr[...] += jnp.sum(oh, axis=1, keepdims=True)

    @pl.when(i == NBLK - 1)
    def _fin():
        pooled = pool_scr[...] / jnp.maximum(cnt_scr[...], 1.0)
        out_ref[...] = (
            jnp.dot(pooled, wl_ref[...], preferred_element_type=jnp.float32)
            + bl_ref[...]
        )


_tc_final = pl.pallas_call(
    _tc_final_body,
    grid=(NBLK,),
    in_specs=[
        pl.BlockSpec((MBLK, HH), lambda i: (i, 0)),
        pl.BlockSpec((MBLK, HH), lambda i: (i, 0)),
        pl.BlockSpec((MBLK, 1), lambda i: (i, 0)),
        pl.BlockSpec((1, HH), lambda i: (0, 0)),
        pl.BlockSpec((1, 1, MBLK), lambda i: (i, 0, 0)),
        pl.BlockSpec((HH, CC), lambda i: (0, 0)),
        pl.BlockSpec((1, CC), lambda i: (0, 0)),
    ],
    out_specs=pl.BlockSpec((GG, CC), lambda i: (0, 0)),
    out_shape=jax.ShapeDtypeStruct((GG, CC), jnp.float32),
    scratch_shapes=[
        pltpu.VMEM((GG, HH), jnp.float32),
        pltpu.VMEM((GG, 1), jnp.float32),
    ],
)


def kernel(x, edge_index, batch, W1, b1, W2, b2, W3, b3, Wl, bl):
    src = edge_index[0].reshape(NS, EPT)
    dst = edge_index[1].reshape(NS, EPT)
    gsrc = jnp.stack([src * 2, src * 2 + 1])
    dst_c = dst.reshape(NS, NCH, CK)
    in0 = dst < NH
    r0 = jnp.where(in0, dst, NH)
    r1 = jnp.where(in0, NH, dst - NH)
    rdst = jnp.stack([r0, r1]).reshape(NC, NS, DNCH, DCK)

    zeros_h = jnp.zeros((RPT, HF), jnp.float32)
    zeros_d = jnp.zeros((DRPT, 16), jnp.float32)
    ones_d = jnp.ones((DCK, 16), jnp.float32)

    deg = _sc_degree(rdst, ones_d, zeros_d)
    deg = deg.reshape(NC, DROWS, 16)
    cnt = jnp.concatenate([deg[0, :NH], deg[1, :NH]], axis=0)
    q1, dinv = _tc_first(cnt, x, W1)

    def agg(q):
        a = _sc_aggregate(q.reshape(2 * NN, HF), gsrc, dst_c, zeros_h)
        return a[:NN].reshape(NN, HH)

    a1 = agg(q1)
    q2 = _tc_mid(a1, q1, dinv, b1.reshape(1, HH), W2)
    a2 = agg(q2)
    q3 = _tc_mid(a2, q2, dinv, b2.reshape(1, HH), W3)
    a3 = agg(q3)
    out = _tc_final(a3, q3, dinv, b3.reshape(1, HH),
                    batch.reshape(NBLK, 1, MBLK), Wl, bl.reshape(1, CC))
    return out


# speedup vs baseline: 1.3604x; 1.2221x over previous
import functools

import jax
import jax.numpy as jnp
from jax import lax
from jax.experimental import pallas as pl
from jax.experimental.pallas import tpu as pltpu
from jax.experimental.pallas import tpu_sc as plsc

NN = 10000
EE = 320000
HH = 128
HF = 64
CC = 10
GG = 64

NC = 2
NS = 16
EPT = EE // NS
CK = 80
NCH = EPT // CK
EPTP = NCH * CK
DDEP = 10
DCK = 80
DNCH = EPT // DCK

ARW = NN
RPT = ARW // NS

NH = NN // 2
DROWS = NH + 8
DRPT = DROWS // NS

MBLK = 1000
NBLK = NN // MBLK

_mesh = plsc.VectorSubcoreMesh(core_axis_name="c", subcore_axis_name="s")


@functools.partial(
    pl.kernel,
    mesh=_mesh,
    compiler_params=pltpu.CompilerParams(use_tc_tiling_on_sc=False),
    out_type=jax.ShapeDtypeStruct((NC, NS, DRPT, 16), jnp.float32),
    scratch_types=[
        pltpu.VMEM((DNCH, DCK), jnp.int32),
        pltpu.VMEM((DCK, 16), jnp.float32),
        pltpu.VMEM((DRPT, 16), jnp.float32),
        pltpu.VMEM_SHARED((DROWS, 16), jnp.float32),
    ] + [pltpu.SemaphoreType.DMA] * DDEP,
)
def _sc_degree(rdst_hbm, ones_hbm, zeros_hbm, out_hbm, didx, ones_v, stage, acc,
               *sems):
    c = lax.axis_index("c")
    s = lax.axis_index("s")
    pltpu.sync_copy(zeros_hbm, stage)
    pltpu.sync_copy(stage, acc.at[pl.ds(s * DRPT, DRPT)])
    pltpu.sync_copy(ones_hbm, ones_v)
    pltpu.sync_copy(rdst_hbm.at[c, s], didx)
    plsc.subcore_barrier()

    def body(k, carry):
        for b in range(DDEP):
            @pl.when(k > 0)
            def _drain():
                pltpu.make_async_copy(ones_hbm, ones_v, sems[b]).wait()

            pltpu.async_copy(ones_v, acc.at[didx.at[DDEP * k + b]], sems[b],
                             add=True)
        return carry

    lax.fori_loop(0, DNCH // DDEP, body, 0)
    for b in range(DDEP):
        pltpu.make_async_copy(ones_hbm, ones_v, sems[b]).wait()
    plsc.subcore_barrier()
    pltpu.sync_copy(acc.at[pl.ds(s * DRPT, DRPT)], stage)
    pltpu.sync_copy(stage, out_hbm.at[c, s])


@functools.partial(
    pl.kernel,
    mesh=_mesh,
    compiler_params=pltpu.CompilerParams(use_tc_tiling_on_sc=False),
    out_type=jax.ShapeDtypeStruct((ARW, NC, HF), jnp.float32),
    scratch_types=[
        pltpu.VMEM((EPTP,), jnp.int32),
        pltpu.VMEM((NCH, CK), jnp.int32),
        pltpu.VMEM((CK, HF), jnp.float32),
        pltpu.VMEM((CK, HF), jnp.float32),
        pltpu.VMEM((RPT, HF), jnp.float32),
        pltpu.VMEM_SHARED((ARW, HF), jnp.float32),
        pltpu.SemaphoreType.DMA,
        pltpu.SemaphoreType.DMA,
    ],
)
def _sc_aggregate(q_hbm, gsrc_hbm, dst_hbm, zeros_hbm, out_hbm,
                  gidx, didx, rows0, rows1, stage, acc, g0, g1):
    c = lax.axis_index("c")
    s = lax.axis_index("s")
    pltpu.sync_copy(zeros_hbm, stage)
    pltpu.sync_copy(stage, acc.at[pl.ds(s * RPT, RPT)])
    pltpu.sync_copy(gsrc_hbm.at[c, s], gidx)
    pltpu.sync_copy(dst_hbm.at[s], didx)
    pltpu.async_copy(q_hbm.at[gidx.at[pl.ds(0, CK)]], rows0, g0)
    plsc.subcore_barrier()

    def body(i2, carry):
        i0 = 2 * i2
        i1 = i0 + 1
        h1 = pltpu.async_copy(q_hbm.at[gidx.at[pl.ds(i1 * CK, CK)]], rows1, g1)
        pltpu.make_async_copy(q_hbm.at[pl.ds(0, CK)], rows0, g0).wait()
        pltpu.sync_copy(rows0, acc.at[didx.at[i0]], add=True)
        inext = jnp.minimum((i0 + 2) * CK, (NCH - 1) * CK)
        pltpu.async_copy(q_hbm.at[gidx.at[pl.ds(inext, CK)]], rows0, g0)
        h1.wait()
        pltpu.sync_copy(rows1, acc.at[didx.at[i1]], add=True)
        return carry

    lax.fori_loop(0, NCH // 2, body, 0)
    pltpu.make_async_copy(q_hbm.at[pl.ds(0, CK)], rows0, g0).wait()
    plsc.subcore_barrier()
    pltpu.sync_copy(acc.at[pl.ds(s * RPT, RPT)], stage)
    pltpu.sync_copy(stage, out_hbm.at[pl.ds(s * RPT, RPT), c])


def _tc_first_body(cnt_ref, x_ref, w_ref, q_ref, dinv_ref):
    deg = cnt_ref[...][:, :1] + 1.0
    dinv = lax.rsqrt(deg)
    p = jnp.dot(x_ref[...], w_ref[...], preferred_element_type=jnp.float32)
    q_ref[...] = p * dinv
    dinv_ref[...] = dinv


_tc_first = pl.pallas_call(
    _tc_first_body,
    grid=(NBLK,),
    in_specs=[
        pl.BlockSpec((MBLK, 16), lambda i: (i, 0)),
        pl.BlockSpec((MBLK, HH), lambda i: (i, 0)),
        pl.BlockSpec((HH, HH), lambda i: (0, 0)),
    ],
    out_specs=[
        pl.BlockSpec((MBLK, HH), lambda i: (i, 0)),
        pl.BlockSpec((MBLK, 1), lambda i: (i, 0)),
    ],
    out_shape=[
        jax.ShapeDtypeStruct((NN, HH), jnp.float32),
        jax.ShapeDtypeStruct((NN, 1), jnp.float32),
    ],
)


def _tc_mid_body(acc_ref, q_ref, dinv_ref, b_ref, w_ref, qn_ref):
    z = (acc_ref[...] + q_ref[...]) * dinv_ref[...] + b_ref[...]
    h = jnp.maximum(z, 0.0)
    qn = jnp.dot(h, w_ref[...], preferred_element_type=jnp.float32)
    qn_ref[...] = qn * dinv_ref[...]


_tc_mid = pl.pallas_call(
    _tc_mid_body,
    grid=(NBLK,),
    in_specs=[
        pl.BlockSpec((MBLK, HH), lambda i: (i, 0)),
        pl.BlockSpec((MBLK, HH), lambda i: (i, 0)),
        pl.BlockSpec((MBLK, 1), lambda i: (i, 0)),
        pl.BlockSpec((1, HH), lambda i: (0, 0)),
        pl.BlockSpec((HH, HH), lambda i: (0, 0)),
    ],
    out_specs=pl.BlockSpec((MBLK, HH), lambda i: (i, 0)),
    out_shape=jax.ShapeDtypeStruct((NN, HH), jnp.float32),
)


def _tc_final_body(acc_ref, q_ref, dinv_ref, b_ref, batch_ref, wl_ref, bl_ref,
                   out_ref, pool_scr, cnt_scr):
    i = pl.program_id(0)

    @pl.when(i == 0)
    def _init():
        pool_scr[...] = jnp.zeros_like(pool_scr)
        cnt_scr[...] = jnp.zeros_like(cnt_scr)

    z = (acc_ref[...] + q_ref[...]) * dinv_ref[...] + b_ref[...]
    h = jnp.maximum(z, 0.0)
    bb = batch_ref[0, 0, :]
    gids = lax.broadcasted_iota(jnp.int32, (GG, MBLK), 0)
    oh = (bb[None, :] == gids).astype(jnp.float32)
    pool_scr[...] += jnp.dot(oh, h, preferred_element_type=jnp.float32)
    cnt_scr[...] += jnp.sum(oh, axis=1, keepdims=True)

    @pl.when(i == NBLK - 1)
    def _fin():
        pooled = pool_scr[...] / jnp.maximum(cnt_scr[...], 1.0)
        out_ref[...] = (
            jnp.dot(pooled, wl_ref[...], preferred_element_type=jnp.float32)
            + bl_ref[...]
        )


_tc_final = pl.pallas_call(
    _tc_final_body,
    grid=(NBLK,),
    in_specs=[
        pl.BlockSpec((MBLK, HH), lambda i: (i, 0)),
        pl.BlockSpec((MBLK, HH), lambda i: (i, 0)),
        pl.BlockSpec((MBLK, 1), lambda i: (i, 0)),
        pl.BlockSpec((1, HH), lambda i: (0, 0)),
        pl.BlockSpec((1, 1, MBLK), lambda i: (i, 0, 0)),
        pl.BlockSpec((HH, CC), lambda i: (0, 0)),
        pl.BlockSpec((1, CC), lambda i: (0, 0)),
    ],
    out_specs=pl.BlockSpec((GG, CC), lambda i: (0, 0)),
    out_shape=jax.ShapeDtypeStruct((GG, CC), jnp.float32),
    scratch_shapes=[
        pltpu.VMEM((GG, HH), jnp.float32),
        pltpu.VMEM((GG, 1), jnp.float32),
    ],
)


def kernel(x, edge_index, batch, W1, b1, W2, b2, W3, b3, Wl, bl):
    src = edge_index[0].reshape(NS, EPT)
    dst = edge_index[1].reshape(NS, EPT)
    gsrc = jnp.stack([src * 2, src * 2 + 1])
    dst_c = dst.reshape(NS, NCH, CK)
    dump = NH + (jnp.arange(EE, dtype=jnp.int32).reshape(NS, EPT) % 8)
    in0 = dst < NH
    r0 = jnp.where(in0, dst, dump)
    r1 = jnp.where(in0, dump, dst - NH)
    rdst = jnp.stack([r0, r1]).reshape(NC, NS, DNCH, DCK)

    zeros_h = jnp.zeros((RPT, HF), jnp.float32)
    zeros_d = jnp.zeros((DRPT, 16), jnp.float32)
    ones_d = jnp.ones((DCK, 16), jnp.float32)

    deg = _sc_degree(rdst, ones_d, zeros_d)
    deg = deg.reshape(NC, DROWS, 16)
    cnt = jnp.concatenate([deg[0, :NH], deg[1, :NH]], axis=0)
    q1, dinv = _tc_first(cnt, x, W1)

    def agg(q):
        a = _sc_aggregate(q.reshape(2 * NN, HF), gsrc, dst_c, zeros_h)
        return a[:NN].reshape(NN, HH)

    a1 = agg(q1)
    q2 = _tc_mid(a1, q1, dinv, b1.reshape(1, HH), W2)
    a2 = agg(q2)
    q3 = _tc_mid(a2, q2, dinv, b2.reshape(1, HH), W3)
    a3 = agg(q3)
    out = _tc_final(a3, q3, dinv, b3.reshape(1, HH),
                    batch.reshape(NBLK, 1, MBLK), Wl, bl.reshape(1, CC))
    return out


# speedup vs baseline: 1.3612x; 1.0006x over previous
import functools

import jax
import jax.numpy as jnp
from jax import lax
from jax.experimental import pallas as pl
from jax.experimental.pallas import tpu as pltpu
from jax.experimental.pallas import tpu_sc as plsc

NN = 10000
EE = 320000
HH = 128
HF = 64
CC = 10
GG = 64

NC = 2
NS = 16
EPT = EE // NS
CK = 80
NCH = EPT // CK
EPTP = NCH * CK
DDEP = 10
DCK = 80
DNCH = EPT // DCK

ARW = NN
RPT = ARW // NS

NH = NN // 2
DROWS = NH + 8
DRPT = DROWS // NS

MBLK = 1000
NBLK = NN // MBLK

_mesh = plsc.VectorSubcoreMesh(core_axis_name="c", subcore_axis_name="s")


@functools.partial(
    pl.kernel,
    mesh=_mesh,
    compiler_params=pltpu.CompilerParams(use_tc_tiling_on_sc=False),
    out_type=jax.ShapeDtypeStruct((NC, NS, DRPT, 16), jnp.float32),
    scratch_types=[
        pltpu.VMEM((DNCH, DCK), jnp.int32),
        pltpu.VMEM((DCK, 16), jnp.float32),
        pltpu.VMEM((DRPT, 16), jnp.float32),
        pltpu.VMEM_SHARED((DROWS, 16), jnp.float32),
    ] + [pltpu.SemaphoreType.DMA] * DDEP,
)
def _sc_degree(rdst_hbm, ones_hbm, zeros_hbm, out_hbm, didx, ones_v, stage, acc,
               *sems):
    c = lax.axis_index("c")
    s = lax.axis_index("s")
    pltpu.sync_copy(zeros_hbm, stage)
    pltpu.sync_copy(stage, acc.at[pl.ds(s * DRPT, DRPT)])
    pltpu.sync_copy(ones_hbm, ones_v)
    pltpu.sync_copy(rdst_hbm.at[c, s], didx)
    plsc.subcore_barrier()

    def body(k, carry):
        for b in range(DDEP):
            @pl.when(k > 0)
            def _drain():
                pltpu.make_async_copy(ones_hbm, ones_v, sems[b]).wait()

            pltpu.async_copy(ones_v, acc.at[didx.at[DDEP * k + b]], sems[b],
                             add=True)
        return carry

    lax.fori_loop(0, DNCH // DDEP, body, 0)
    for b in range(DDEP):
        pltpu.make_async_copy(ones_hbm, ones_v, sems[b]).wait()
    plsc.subcore_barrier()
    pltpu.sync_copy(acc.at[pl.ds(s * DRPT, DRPT)], stage)
    pltpu.sync_copy(stage, out_hbm.at[c, s])


@functools.partial(
    pl.kernel,
    mesh=_mesh,
    compiler_params=pltpu.CompilerParams(use_tc_tiling_on_sc=False),
    out_type=jax.ShapeDtypeStruct((ARW, NC, HF), jnp.float32),
    scratch_types=[
        pltpu.VMEM((EPTP,), jnp.int32),
        pltpu.VMEM((NCH, CK), jnp.int32),
        pltpu.VMEM((CK, HF), jnp.float32),
        pltpu.VMEM((CK, HF), jnp.float32),
        pltpu.VMEM((RPT, HF), jnp.float32),
        pltpu.VMEM_SHARED((ARW, HF), jnp.float32),
        pltpu.SemaphoreType.DMA,
        pltpu.SemaphoreType.DMA,
    ],
)
def _sc_aggregate(q_hbm, gsrc_hbm, dst_hbm, zeros_hbm, out_hbm,
                  gidx, didx, rows0, rows1, stage, acc, g0, g1):
    c = lax.axis_index("c")
    s = lax.axis_index("s")
    pltpu.sync_copy(zeros_hbm, stage)
    pltpu.sync_copy(stage, acc.at[pl.ds(s * RPT, RPT)])
    pltpu.sync_copy(gsrc_hbm.at[c, s], gidx)
    pltpu.sync_copy(dst_hbm.at[s], didx)
    pltpu.async_copy(q_hbm.at[gidx.at[pl.ds(0, CK)]], rows0, g0)
    plsc.subcore_barrier()

    def body(i2, carry):
        i0 = 2 * i2
        i1 = i0 + 1
        h1 = pltpu.async_copy(q_hbm.at[gidx.at[pl.ds(i1 * CK, CK)]], rows1, g1)
        pltpu.make_async_copy(q_hbm.at[pl.ds(0, CK)], rows0, g0).wait()
        pltpu.sync_copy(rows0, acc.at[didx.at[i0]], add=True)
        inext = jnp.minimum((i0 + 2) * CK, (NCH - 1) * CK)
        pltpu.async_copy(q_hbm.at[gidx.at[pl.ds(inext, CK)]], rows0, g0)
        h1.wait()
        pltpu.sync_copy(rows1, acc.at[didx.at[i1]], add=True)
        return carry

    lax.fori_loop(0, NCH // 2, body, 0)
    pltpu.make_async_copy(q_hbm.at[pl.ds(0, CK)], rows0, g0).wait()
    plsc.subcore_barrier()
    pltpu.sync_copy(acc.at[pl.ds(s * RPT, RPT)], stage)
    pltpu.sync_copy(stage, out_hbm.at[pl.ds(s * RPT, RPT), c])


def _tc_first_body(cnt_ref, x_ref, w_ref, q_ref, dinv_ref):
    deg = cnt_ref[...][:, :1] + 1.0
    dinv = lax.rsqrt(deg)
    p = jnp.dot(x_ref[...], w_ref[...], preferred_element_type=jnp.float32)
    q_ref[...] = p * dinv
    dinv_ref[...] = dinv


_tc_first = pl.pallas_call(
    _tc_first_body,
    grid=(NBLK,),
    in_specs=[
        pl.BlockSpec((MBLK, 16), lambda i: (i, 0)),
        pl.BlockSpec((MBLK, HH), lambda i: (i, 0)),
        pl.BlockSpec((HH, HH), lambda i: (0, 0)),
    ],
    out_specs=[
        pl.BlockSpec((MBLK, HH), lambda i: (i, 0)),
        pl.BlockSpec((MBLK, 1), lambda i: (i, 0)),
    ],
    out_shape=[
        jax.ShapeDtypeStruct((NN, HH), jnp.float32),
        jax.ShapeDtypeStruct((NN, 1), jnp.float32),
    ],
)


def _tc_mid_body(acc_ref, q_ref, dinv_ref, b_ref, w_ref, qn_ref):
    z = (acc_ref[...] + q_ref[...]) * dinv_ref[...] + b_ref[...]
    h = jnp.maximum(z, 0.0)
    qn = jnp.dot(h, w_ref[...], preferred_element_type=jnp.float32)
    qn_ref[...] = qn * dinv_ref[...]


_tc_mid = pl.pallas_call(
    _tc_mid_body,
    grid=(NBLK,),
    in_specs=[
        pl.BlockSpec((MBLK, HH), lambda i: (i, 0)),
        pl.BlockSpec((MBLK, HH), lambda i: (i, 0)),
        pl.BlockSpec((MBLK, 1), lambda i: (i, 0)),
        pl.BlockSpec((1, HH), lambda i: (0, 0)),
        pl.BlockSpec((HH, HH), lambda i: (0, 0)),
    ],
    out_specs=pl.BlockSpec((MBLK, HH), lambda i: (i, 0)),
    out_shape=jax.ShapeDtypeStruct((NN, HH), jnp.float32),
)


def _tc_final_body(acc_ref, q_ref, dinv_ref, b_ref, batch_ref, wl_ref, bl_ref,
                   out_ref, pool_scr, cnt_scr):
    i = pl.program_id(0)

    @pl.when(i == 0)
    def _init():
        pool_scr[...] = jnp.zeros_like(pool_scr)
        cnt_scr[...] = jnp.zeros_like(cnt_scr)

    z = (acc_ref[...] + q_ref[...]) * dinv_ref[...] + b_ref[...]
    h = jnp.maximum(z, 0.0)
    bb = batch_ref[0, 0, :]
    gids = lax.broadcasted_iota(jnp.int32, (GG, MBLK), 0)
    oh = (bb[None, :] == gids).astype(jnp.float32)
    pool_scr[...] += jnp.dot(oh, h, preferred_element_type=jnp.float32)
    cnt_scr[...] += jnp.sum(oh, axis=1, keepdims=True)

    @pl.when(i == NBLK - 1)
    def _fin():
        pooled = pool_scr[...] / jnp.maximum(cnt_scr[...], 1.0)
        out_ref[...] = (
            jnp.dot(pooled, wl_ref[...], preferred_element_type=jnp.float32)
            + bl_ref[...]
        )


_tc_final = pl.pallas_call(
    _tc_final_body,
    grid=(NBLK,),
    in_specs=[
        pl.BlockSpec((MBLK, HH), lambda i: (i, 0)),
        pl.BlockSpec((MBLK, HH), lambda i: (i, 0)),
        pl.BlockSpec((MBLK, 1), lambda i: (i, 0)),
        pl.BlockSpec((1, HH), lambda i: (0, 0)),
        pl.BlockSpec((1, 1, MBLK), lambda i: (i, 0, 0)),
        pl.BlockSpec((HH, CC), lambda i: (0, 0)),
        pl.BlockSpec((1, CC), lambda i: (0, 0)),
    ],
    out_specs=pl.BlockSpec((GG, CC), lambda i: (0, 0)),
    out_shape=jax.ShapeDtypeStruct((GG, CC), jnp.float32),
    scratch_shapes=[
        pltpu.VMEM((GG, HH), jnp.float32),
        pltpu.VMEM((GG, 1), jnp.float32),
    ],
)


def kernel(x, edge_index, batch, W1, b1, W2, b2, W3, b3, Wl, bl):
    src = edge_index[0].reshape(NS, EPT)
    dst = edge_index[1].reshape(NS, EPT)
    gsrc = jnp.stack([src * 2, src * 2 + 1])
    dst_c = dst.reshape(NS, NCH, CK)
    dump = NH + (dst & 7)
    in0 = dst < NH
    r0 = jnp.where(in0, dst, dump)
    r1 = jnp.where(in0, dump, dst - NH)
    rdst = jnp.stack([r0, r1]).reshape(NC, NS, DNCH, DCK)

    zeros_h = jnp.zeros((RPT, HF), jnp.float32)
    zeros_d = jnp.zeros((DRPT, 16), jnp.float32)
    ones_d = jnp.ones((DCK, 16), jnp.float32)

    deg = _sc_degree(rdst, ones_d, zeros_d)
    deg = deg.reshape(NC, DROWS, 16)
    cnt = jnp.concatenate([deg[0, :NH], deg[1, :NH]], axis=0)
    q1, dinv = _tc_first(cnt, x, W1)

    def agg(q):
        a = _sc_aggregate(q.reshape(2 * NN, HF), gsrc, dst_c, zeros_h)
        return a[:NN].reshape(NN, HH)

    a1 = agg(q1)
    q2 = _tc_mid(a1, q1, dinv, b1.reshape(1, HH), W2)
    a2 = agg(q2)
    q3 = _tc_mid(a2, q2, dinv, b2.reshape(1, HH), W3)
    a3 = agg(q3)
    out = _tc_final(a3, q3, dinv, b3.reshape(1, HH),
                    batch.reshape(NBLK, 1, MBLK), Wl, bl.reshape(1, CC))
    return out
